# baseline (speedup 1.0000x reference)
import functools

import jax
import jax.numpy as jnp
from jax import lax
from jax.experimental import pallas as pl
from jax.experimental.pallas import tpu as pltpu
from jax.experimental.pallas import tpu_sc as plsc

NUM_NODES = 10000
NUM_RELATIONS = 4
E_PER_REL = 80000
D_FEAT = 128
D_EDGE = 16
D_HIDDEN = 128
LN_EPS = 1e-6

NC = 2
NS = 16
CHUNK = 64
N_CHUNKS = E_PER_REL // CHUNK
MAIN_G = N_CHUNKS // NS // 2
N_TAIL = N_CHUNKS - 2 * MAIN_G * NS
N_PAD = 10240
ROWS_PER_TILE = N_PAD // NS
AW = D_FEAT + 2 * D_EDGE
DEG_COL = D_FEAT + D_EDGE


def _fill(ref, val):
    rows, width = ref.shape

    def body(i, carry):
        for j in range(width // 16):
            ref[i, pl.ds(j * 16, 16)] = jnp.full((16,), val, ref.dtype)
        return carry

    lax.fori_loop(0, rows, body, 0)


def _sc_segment_sums(nodes_aug, senders, receivers, edges):
    mesh = plsc.VectorSubcoreMesh(core_axis_name="c", subcore_axis_name="s",
                                  num_cores=NC, num_subcores=NS)

    @functools.partial(
        pl.kernel,
        out_type=jax.ShapeDtypeStruct((NUM_RELATIONS, N_PAD, AW),
                                      jnp.float32),
        mesh=mesh,
        scratch_types=[
            pltpu.VMEM_SHARED((N_PAD, AW), jnp.float32),
            pltpu.VMEM((2, CHUNK), jnp.int32),
            pltpu.VMEM((2, CHUNK), jnp.int32),
            pltpu.VMEM((CHUNK, AW), jnp.float32),
            pltpu.VMEM((CHUNK, AW), jnp.float32),
            pltpu.VMEM((CHUNK, D_EDGE), jnp.float32),
            pltpu.VMEM((CHUNK, D_EDGE), jnp.float32),
            pltpu.SemaphoreType.DMA,
            pltpu.SemaphoreType.DMA,
            pltpu.SemaphoreType.DMA,
            pltpu.SemaphoreType.DMA,
            pltpu.SemaphoreType.DMA,
        ],
        compiler_params=pltpu.CompilerParams(use_tc_tiling_on_sc=False),
    )
    def sc_kernel(nodes_hbm, sr_hbm, edges_hbm, acc_out,
                  acc_sh, idx0, idx1, rows0, rows1, er0, er1,
                  sem0, sem1, se0, se1, sem_s):
        cid = lax.axis_index("c")
        sid = lax.axis_index("s")
        stripe = pl.ds(pl.multiple_of(sid * ROWS_PER_TILE, 8), ROWS_PER_TILE)

        for phase in range(2):
            r = cid + 2 * phase
            use_edges = phase == 0

            def load(k, idx, er, sem_e):
                pltpu.sync_copy(sr_hbm.at[r].at[k], idx)
                if use_edges:
                    base = pl.ds(pl.multiple_of(k * CHUNK, CHUNK), CHUNK)
                    pltpu.async_copy(edges_hbm.at[r].at[base], er, sem_e)

            def gather(idx, rows, sem):
                pltpu.async_copy(nodes_hbm.at[idx.at[0]], rows, sem)

            def consume(k, idx, rows, er, sem, sem_e):
                pltpu.make_async_copy(nodes_hbm.at[idx.at[0]], rows,
                                      sem).wait()
                if use_edges:
                    base = pl.ds(pl.multiple_of(k * CHUNK, CHUNK), CHUNK)
                    pltpu.make_async_copy(edges_hbm.at[r].at[base], er,
                                          sem_e).wait()
                    for j in range(CHUNK):
                        rows[j, pl.ds(D_FEAT, D_EDGE)] = er[j,
                                                            pl.ds(0, D_EDGE)]
                pltpu.async_copy(rows, acc_sh.at[idx.at[1]], sem_s, add=True)
                pltpu.make_async_copy(rows, acc_sh.at[idx.at[1]],
                                      sem_s).wait()

            _fill(rows0, 0.0)

            def zero_body(k, carry):
                dst = pl.ds(
                    pl.multiple_of(sid * ROWS_PER_TILE + k * CHUNK, 8), CHUNK)
                pltpu.sync_copy(rows0, acc_sh.at[dst])
                return carry

            lax.fori_loop(0, ROWS_PER_TILE // CHUNK, zero_body, 0)

            load(sid, idx0, er0, se0)
            gather(idx0, rows0, sem0)
            plsc.subcore_barrier()

            def body(g, carry):
                k1 = (2 * g + 1) * NS + sid
                load(k1, idx1, er1, se1)
                gather(idx1, rows1, sem1)
                k0 = 2 * g * NS + sid
                consume(k0, idx0, rows0, er0, sem0, se0)

                @pl.when(jnp.logical_or(g < MAIN_G - 1, sid < N_TAIL))
                def _():
                    k2 = (2 * g + 2) * NS + sid
                    load(k2, idx0, er0, se0)
                    gather(idx0, rows0, sem0)

                consume(k1, idx1, rows1, er1, sem1, se1)
                return carry

            lax.fori_loop(0, MAIN_G, body, 0)

            @pl.when(sid < N_TAIL)
            def _():
                kt = 2 * MAIN_G * NS + sid
                consume(kt, idx0, rows0, er0, sem0, se0)

            plsc.subcore_barrier()
            pltpu.sync_copy(acc_sh.at[stripe], acc_out.at[r].at[stripe])

    sr = jnp.stack([senders.reshape(NUM_RELATIONS, N_CHUNKS, CHUNK),
                    receivers.reshape(NUM_RELATIONS, N_CHUNKS, CHUNK)],
                   axis=2)
    return sc_kernel(nodes_aug, sr, edges)


BR = 1000


def _tc_dense(nodes, acc, W_aug, ln_scale, ln_bias, W_node):
    def body(nodes_ref, acc_ref, wa_ref, wn_ref, g_ref, b_ref, o_ref):
        x = jnp.dot(nodes_ref[...], wn_ref[...],
                    preferred_element_type=jnp.float32)
        for r in range(NUM_RELATIONS):
            m = jnp.dot(acc_ref[r], wa_ref[r],
                        preferred_element_type=jnp.float32)
            d = acc_ref[r, :, DEG_COL:DEG_COL + 1]
            x = x + m * lax.reciprocal(jnp.maximum(d, 1.0))
        mean = jnp.mean(x, axis=-1, keepdims=True)
        var = jnp.mean(jnp.square(x - mean), axis=-1, keepdims=True)
        x = (x - mean) * lax.rsqrt(var + LN_EPS) * g_ref[...] + b_ref[...]
        o_ref[...] = jnp.maximum(x, 0.0)

    grid = (NUM_NODES // BR,)
    return pl.pallas_call(
        body,
        grid=grid,
        in_specs=[
            pl.BlockSpec((BR, D_FEAT), lambda i: (i, 0)),
            pl.BlockSpec((NUM_RELATIONS, BR, AW), lambda i: (0, i, 0)),
            pl.BlockSpec((NUM_RELATIONS, AW, D_HIDDEN), lambda i: (0, 0, 0)),
            pl.BlockSpec((D_FEAT, D_HIDDEN), lambda i: (0, 0)),
            pl.BlockSpec((1, D_HIDDEN), lambda i: (0, 0)),
            pl.BlockSpec((1, D_HIDDEN), lambda i: (0, 0)),
        ],
        out_specs=pl.BlockSpec((BR, D_HIDDEN), lambda i: (i, 0)),
        out_shape=jax.ShapeDtypeStruct((NUM_NODES, D_HIDDEN), jnp.float32),
    )(nodes, acc, W_aug, W_node,
      ln_scale.reshape(1, D_HIDDEN), ln_bias.reshape(1, D_HIDDEN))


def kernel(nodes, edges, senders, receivers, W_node, W_rel0, W_rel1, W_rel2,
           W_rel3, ln_scale, ln_bias):
    nodes_aug = jnp.concatenate(
        [nodes,
         jnp.zeros((NUM_NODES, D_EDGE), jnp.float32),
         jnp.ones((NUM_NODES, D_EDGE), jnp.float32)], axis=1)
    acc = _sc_segment_sums(nodes_aug, senders, receivers, edges)
    zpad = jnp.zeros((D_EDGE, D_HIDDEN), jnp.float32)
    W_aug = jnp.stack([
        jnp.concatenate([W_rel0, zpad], axis=0),
        jnp.concatenate([W_rel1, zpad], axis=0),
        jnp.concatenate([W_rel2, zpad, zpad], axis=0),
        jnp.concatenate([W_rel3, zpad, zpad], axis=0)], axis=0)
    return _tc_dense(nodes, acc, W_aug, ln_scale, ln_bias, W_node)
